# baseline (speedup 1.0000x reference)
import functools

import jax
import jax.numpy as jnp
from jax import lax
from jax.experimental import pallas as pl
from jax.experimental.pallas import tpu as pltpu
from jax.experimental.pallas import tpu_sc as plsc

E = 64
D = 768
TB = 1024
NW = 32
C = 512
L = 16


def _tc_logits_body(x_ref, w_ref, b_ref, lg_ref):
    xb = x_ref[...]
    w = w_ref[...]
    lg_ref[...] = jax.lax.dot_general(
        w, xb, (((1,), (1,)), ((), ())),
        preferred_element_type=jnp.float32) + b_ref[...]


def _sc_route_body(lg_hbm, rw_hbm, idx_hbm, lv, rv, iv):
    wid = lax.axis_index("s") * 2 + lax.axis_index("c")
    lane = lax.iota(jnp.int32, L)
    neg_inf = jnp.full((L,), -jnp.inf, dtype=jnp.float32)
    zero_i = jnp.zeros((L,), dtype=jnp.int32)
    zeros_f = jnp.zeros((L,), dtype=jnp.float32)

    for ch in range(2):
        tbase = wid * (2 * C) + ch * C

        pltpu.sync_copy(lg_hbm.at[:, pl.ds(tbase, C)], lv)

        def zbody(j, _):
            rv[pl.ds(j * L, L)] = zeros_f
            return 0
        lax.fori_loop(0, C * E // L, zbody, 0)

        def gbody(g, _):
            goff = g * L
            m1, i1 = neg_inf, zero_i
            m2, i2 = neg_inf, zero_i
            for e in range(E):
                v = lv[e, pl.ds(goff, L)]
                es = jnp.full((L,), e, dtype=jnp.int32)
                gt1 = v > m1
                gt2 = v > m2
                nm2 = jnp.where(gt1, m1, jnp.where(gt2, v, m2))
                ni2 = jnp.where(gt1, i1, jnp.where(gt2, es, i2))
                m1 = jnp.where(gt1, v, m1)
                i1 = jnp.where(gt1, es, i1)
                m2, i2 = nm2, ni2
            ew = jnp.exp(m2 - m1)
            s = 1.0 / (1.0 + ew)
            w1 = s
            w2 = ew * s
            tok = goff + lane
            plsc.store_scatter(rv, [tok * E + i1], w1)
            plsc.store_scatter(rv, [tok * E + i2], w2)
            plsc.store_scatter(iv, [tok * 2], i1)
            plsc.store_scatter(iv, [tok * 2 + 1], i2)
            return 0
        lax.fori_loop(0, C // L, gbody, 0)

        pltpu.sync_copy(rv, rw_hbm.at[pl.ds(tbase * E, C * E)])
        pltpu.sync_copy(iv, idx_hbm.at[pl.ds(tbase * 2, C * 2)])


def kernel(x, W, b):
    T = x.shape[0] * x.shape[1]
    xf = x.reshape(T, D)
    b2 = b.reshape(E, 1)

    logits = pl.pallas_call(
        _tc_logits_body,
        grid=(T // TB,),
        in_specs=[
            pl.BlockSpec((TB, D), lambda i: (i, 0)),
            pl.BlockSpec((E, D), lambda i: (0, 0)),
            pl.BlockSpec((E, 1), lambda i: (0, 0)),
        ],
        out_specs=pl.BlockSpec((E, TB), lambda i: (0, i)),
        out_shape=jax.ShapeDtypeStruct((E, T), jnp.float32),
    )(xf, W, b2)

    mesh = plsc.VectorSubcoreMesh(core_axis_name="c", subcore_axis_name="s")
    route = functools.partial(
        pl.kernel,
        out_type=[
            jax.ShapeDtypeStruct((T * E,), jnp.float32),
            jax.ShapeDtypeStruct((T * 2,), jnp.int32),
        ],
        mesh=mesh,
        compiler_params=pltpu.CompilerParams(needs_layout_passes=False),
        scratch_types=[
            pltpu.VMEM((E, C), jnp.float32),
            pltpu.VMEM((C * E,), jnp.float32),
            pltpu.VMEM((C * 2,), jnp.int32),
        ],
    )(_sc_route_body)
    rw_flat, idx_flat = route(logits)
    return (rw_flat.reshape(T, E), idx_flat.reshape(T, 2))


# speedup vs baseline: 2.2161x; 2.2161x over previous
import jax
import jax.numpy as jnp
from jax.experimental import pallas as pl

E = 64
D = 768
TB = 1024


def _tc_router_body(x_ref, w_ref, b_ref, rw_ref, idx_ref):
    xb = x_ref[...]
    w = w_ref[...]
    logits = jax.lax.dot_general(
        xb, w, (((1,), (1,)), ((), ())),
        preferred_element_type=jnp.float32) + b_ref[...]
    iota = jax.lax.broadcasted_iota(jnp.int32, (TB, E), 1)
    m1 = jnp.max(logits, axis=1, keepdims=True)
    i1 = jnp.min(jnp.where(logits == m1, iota, E), axis=1, keepdims=True)
    masked = jnp.where(iota == i1, -jnp.inf, logits)
    m2 = jnp.max(masked, axis=1, keepdims=True)
    i2 = jnp.min(jnp.where(masked == m2, iota, E), axis=1, keepdims=True)
    ew = jnp.exp(m2 - m1)
    s = 1.0 / (1.0 + ew)
    w1 = s
    w2 = ew * s
    rw = (jnp.where(iota == i1, w1, 0.0) + jnp.where(iota == i2, w2, 0.0))
    rw_ref[...] = rw.T
    iota2 = jax.lax.broadcasted_iota(jnp.int32, (2, TB), 0)
    idx_ref[...] = jnp.where(iota2 == 0, i1.T, i2.T)


def kernel(x, W, b):
    T = x.shape[0] * x.shape[1]
    xf = x.reshape(T, D)
    b2 = b.reshape(1, E)
    grid = (T // TB,)
    rw_t, idx_t = pl.pallas_call(
        _tc_router_body,
        grid=grid,
        in_specs=[
            pl.BlockSpec((TB, D), lambda i: (i, 0)),
            pl.BlockSpec((E, D), lambda i: (0, 0)),
            pl.BlockSpec((1, E), lambda i: (0, 0)),
        ],
        out_specs=[
            pl.BlockSpec((E, TB), lambda i: (0, i)),
            pl.BlockSpec((2, TB), lambda i: (0, i)),
        ],
        out_shape=[
            jax.ShapeDtypeStruct((E, T), jnp.float32),
            jax.ShapeDtypeStruct((2, T), jnp.int32),
        ],
    )(xf, W, b2)
    return (rw_t.T, idx_t.T)
